# baseline (speedup 1.0000x reference)
import functools

import jax
import jax.numpy as jnp
from jax import lax
from jax.experimental import pallas as pl
from jax.experimental.pallas import tpu as pltpu
from jax.experimental.pallas import tpu_sc as plsc

_N = 10000
_E = 320000
_DIN = 128
_C = 101
_OUT = 200
_L = 6
_G = 256
_F = 256
_BLK = 2000
_K = 96
_CHUNKS0 = 204
_CHUNKS1 = 6
_NCK = 16 * (_CHUNKS0 + _CHUNKS1)
_EPAD = _NCK * _K
_AROWS = _N + 16
_RPT = _N // 16


_sc_mesh = plsc.VectorSubcoreMesh(core_axis_name="c", subcore_axis_name="s")


@functools.partial(
    pl.kernel,
    out_type=jax.ShapeDtypeStruct((2, _N, _F), jnp.bfloat16),
    mesh=_sc_mesh,
    compiler_params=pltpu.CompilerParams(use_tc_tiling_on_sc=False),
    scratch_types=[
        [[pltpu.VMEM((2, _K), jnp.int32)] * 2] * 3,
        [pltpu.VMEM((_K, _F), jnp.bfloat16)] * 3,
        pltpu.VMEM_SHARED((_AROWS, _F), jnp.bfloat16),
        [[pltpu.SemaphoreType.DMA] * 2] * 3,
        [pltpu.SemaphoreType.DMA] * 3,
    ],
)
def _segsum(m_hbm, idx_hbm, zc_hbm, out_hbm, ib, rows, agg_sh, si, sg):
    cid = lax.axis_index("c")
    sid = lax.axis_index("s")

    def _prologue(c0):
        for j in range(6):
            b, tp = j % 3, j // 3
            pltpu.async_copy(idx_hbm.at[c0 + j], ib[b][tp], si[b][tp])
        for j in range(3):
            pltpu.make_async_copy(idx_hbm.at[c0 + j], ib[j][0],
                                  si[j][0]).wait()
            pltpu.async_copy(m_hbm.at[ib[j][0].at[0]], rows[j], sg[j])

    def _steady(c, b, tp):
        ntp = 1 - tp
        pltpu.make_async_copy(m_hbm.at[ib[b][tp].at[0]], rows[b],
                              sg[b]).wait()
        pltpu.sync_copy(rows[b], agg_sh.at[ib[b][tp].at[1]], add=True)
        pltpu.async_copy(idx_hbm.at[c + 6], ib[b][tp], si[b][tp])
        pltpu.make_async_copy(idx_hbm.at[c + 3], ib[b][ntp],
                              si[b][ntp]).wait()
        pltpu.async_copy(m_hbm.at[ib[b][ntp].at[0]], rows[b], sg[b])

    def _drain(c0, nchunks):
        def _ring(t, carry):
            base = c0 + t * 6
            for j in range(6):
                _steady(base + j, j % 3, j // 3)
            return carry

        lax.fori_loop(0, (nchunks - 6) // 6, _ring, 0)
        for j in range(3):
            c = c0 + nchunks - 6 + j
            pltpu.make_async_copy(m_hbm.at[ib[j][0].at[0]], rows[j],
                                  sg[j]).wait()
            pltpu.sync_copy(rows[j], agg_sh.at[ib[j][0].at[1]], add=True)
            pltpu.make_async_copy(idx_hbm.at[c + 3], ib[j][1],
                                  si[j][1]).wait()
            pltpu.async_copy(m_hbm.at[ib[j][1].at[0]], rows[j], sg[j])
        for j in range(3):
            pltpu.make_async_copy(m_hbm.at[ib[j][1].at[0]], rows[j],
                                  sg[j]).wait()
            pltpu.sync_copy(rows[j], agg_sh.at[ib[j][1].at[1]], add=True)

    @pl.when(cid == 0)
    def _():
        _prologue(sid * _CHUNKS0)

    @pl.when(cid == 1)
    def _():
        _prologue(16 * _CHUNKS0 + sid * _CHUNKS1)

    r0 = sid * _RPT
    pltpu.sync_copy(zc_hbm.at[cid * 16 + sid], agg_sh.at[pl.ds(r0, _RPT)])
    plsc.subcore_barrier()

    @pl.when(cid == 0)
    def _():
        _drain(sid * _CHUNKS0, _CHUNKS0)

    @pl.when(cid == 1)
    def _():
        _drain(16 * _CHUNKS0 + sid * _CHUNKS1, _CHUNKS1)

    plsc.subcore_barrier()
    pltpu.sync_copy(agg_sh.at[pl.ds(r0, _RPT)],
                    out_hbm.at[cid, pl.ds(r0, _RPT), :])


def _encode_body(x_ref, we_ref, be_ref, w0_ref, h_ref, m0_ref):
    h = jnp.maximum(
        jnp.dot(x_ref[...], we_ref[...], preferred_element_type=jnp.float32)
        + be_ref[...], 0.0)
    h_ref[...] = h
    m = jnp.dot(h, w0_ref[...], preferred_element_type=jnp.float32)
    m0_ref[...] = m.astype(jnp.bfloat16)


_encode = pl.pallas_call(
    _encode_body,
    grid=(_N // _BLK,),
    in_specs=[
        pl.BlockSpec((_BLK, _DIN), lambda i: (i, 0)),
        pl.BlockSpec((_DIN, _OUT), lambda i: (0, 0)),
        pl.BlockSpec((1, _OUT), lambda i: (0, 0)),
        pl.BlockSpec((_OUT, _F), lambda i: (0, 0)),
    ],
    out_specs=[
        pl.BlockSpec((_BLK, _OUT), lambda i: (i, 0)),
        pl.BlockSpec((_BLK, _F), lambda i: (i, 0)),
    ],
    out_shape=[
        jax.ShapeDtypeStruct((_N, _OUT), jnp.float32),
        jax.ShapeDtypeStruct((_N, _F), jnp.bfloat16),
    ],
)


def _gru_body(a_ref, h_ref, wir, wiz, win, whr, whz, whn,
              bir, biz, bin_, bhr, bhz, bhn, wnx, ho_ref, mo_ref):
    f32 = jnp.float32
    av = a_ref[...]
    a = av[0].astype(f32) + av[1].astype(f32)
    h = h_ref[...]
    i_r = jnp.dot(a, wir[...], preferred_element_type=f32) + bir[...]
    i_z = jnp.dot(a, wiz[...], preferred_element_type=f32) + biz[...]
    i_n = jnp.dot(a, win[...], preferred_element_type=f32) + bin_[...]
    h_r = jnp.dot(h, whr[...], preferred_element_type=f32) + bhr[...]
    h_z = jnp.dot(h, whz[...], preferred_element_type=f32) + bhz[...]
    h_n = jnp.dot(h, whn[...], preferred_element_type=f32) + bhn[...]
    r = jax.nn.sigmoid(i_r + h_r)
    z = jax.nn.sigmoid(i_z + h_z)
    n = jnp.tanh(i_n + r * h_n)
    hn = (1.0 - z) * n + z * h
    ho_ref[...] = hn
    mn = jnp.dot(hn, wnx[...], preferred_element_type=f32)
    mo_ref[...] = mn.astype(jnp.bfloat16)


_gru = pl.pallas_call(
    _gru_body,
    grid=(_N // _BLK,),
    in_specs=[
        pl.BlockSpec((2, _BLK, _F), lambda i: (0, i, 0)),
        pl.BlockSpec((_BLK, _OUT), lambda i: (i, 0)),
    ] + [pl.BlockSpec((_F, _OUT), lambda i: (0, 0))] * 3
      + [pl.BlockSpec((_OUT, _OUT), lambda i: (0, 0))] * 3
      + [pl.BlockSpec((1, _OUT), lambda i: (0, 0))] * 6
      + [pl.BlockSpec((_OUT, _F), lambda i: (0, 0))],
    out_specs=[
        pl.BlockSpec((_BLK, _OUT), lambda i: (i, 0)),
        pl.BlockSpec((_BLK, _F), lambda i: (i, 0)),
    ],
    out_shape=[
        jax.ShapeDtypeStruct((_N, _OUT), jnp.float32),
        jax.ShapeDtypeStruct((_N, _F), jnp.bfloat16),
    ],
)


def _pool_body(h_ref, hc_ref, b_ref, w1a, w1b, b1, w2, b2, w3, b3, o_ref):
    f32 = jnp.float32
    bvec = b_ref[...]
    gids = lax.broadcasted_iota(jnp.int32, (_G, _N), 0)
    onehot = jnp.where(bvec == gids, 1.0, 0.0).astype(f32)
    cnt = jnp.sum(onehot, axis=1, keepdims=True)
    inv = 1.0 / jnp.maximum(cnt, 1.0)
    gr_h = jnp.dot(onehot, h_ref[...], preferred_element_type=f32) * inv
    gr_c = jnp.dot(onehot, hc_ref[...], preferred_element_type=f32) * inv
    h1 = jnp.maximum(
        jnp.dot(gr_h, w1a[...], preferred_element_type=f32)
        + jnp.dot(gr_c, w1b[...], preferred_element_type=f32)
        + b1[...], 0.0)
    h2 = jnp.maximum(jnp.dot(h1, w2[...], preferred_element_type=f32)
                     + b2[...], 0.0)
    o_ref[...] = jax.nn.sigmoid(
        jnp.dot(h2, w3[...], preferred_element_type=f32) + b3[...])


_pool = pl.pallas_call(
    _pool_body,
    in_specs=[
        pl.BlockSpec((_N, _OUT), lambda: (0, 0)),
        pl.BlockSpec((_N, _OUT), lambda: (0, 0)),
        pl.BlockSpec((1, _N), lambda: (0, 0)),
        pl.BlockSpec((_OUT, 256), lambda: (0, 0)),
        pl.BlockSpec((_OUT, 256), lambda: (0, 0)),
        pl.BlockSpec((1, 256), lambda: (0, 0)),
        pl.BlockSpec((256, 128), lambda: (0, 0)),
        pl.BlockSpec((1, 128), lambda: (0, 0)),
        pl.BlockSpec((128, 1), lambda: (0, 0)),
        pl.BlockSpec((1, 1), lambda: (0, 0)),
    ],
    out_specs=pl.BlockSpec((_G, 1), lambda: (0, 0)),
    out_shape=jax.ShapeDtypeStruct((_G, 1), jnp.float32),
)


def kernel(x, edge_index, batch, W_enc, b_enc, weight, W_ih, W_hh,
           b_ih, b_hh, W1, b1, W2, b2, W3, b3):
    f32 = jnp.float32
    src = edge_index[0]
    dst = edge_index[1]
    zc = jnp.zeros((32, _RPT, _F), jnp.bfloat16)
    npad = _EPAD - _E
    srcp = jnp.concatenate([src, jnp.zeros((npad,), jnp.int32)])
    dstp = jnp.concatenate([dst, jnp.full((npad,), _N, jnp.int32)])
    nck = _EPAD // _K
    idx = jnp.stack([srcp.reshape(nck, _K), dstp.reshape(nck, _K)], axis=1)

    W_encp = jnp.pad(W_enc, ((0, 0), (0, _OUT - _C)))
    b_encp = jnp.pad(b_enc, (0, _OUT - _C)).reshape(1, _OUT)
    wpad = jnp.pad(weight, ((0, 0), (0, 0), (0, _F - _OUT)))
    ihT = W_ih.T
    hhT = W_hh.T
    pad_f = ((0, _F - _OUT), (0, 0))
    wir = jnp.pad(ihT[:, 0 * _OUT:1 * _OUT], pad_f)
    wiz = jnp.pad(ihT[:, 1 * _OUT:2 * _OUT], pad_f)
    win = jnp.pad(ihT[:, 2 * _OUT:3 * _OUT], pad_f)
    whr = hhT[:, 0 * _OUT:1 * _OUT]
    whz = hhT[:, 1 * _OUT:2 * _OUT]
    whn = hhT[:, 2 * _OUT:3 * _OUT]
    bir = b_ih[0 * _OUT:1 * _OUT].reshape(1, _OUT)
    biz = b_ih[1 * _OUT:2 * _OUT].reshape(1, _OUT)
    bin_ = b_ih[2 * _OUT:3 * _OUT].reshape(1, _OUT)
    bhr = b_hh[0 * _OUT:1 * _OUT].reshape(1, _OUT)
    bhz = b_hh[1 * _OUT:2 * _OUT].reshape(1, _OUT)
    bhn = b_hh[2 * _OUT:3 * _OUT].reshape(1, _OUT)
    W1a = W1[:_OUT]
    W1b = jnp.pad(W1[_OUT:], ((0, 2 * _OUT - W1.shape[0]), (0, 0)))
    b1r = b1.reshape(1, 256)
    b2r = b2.reshape(1, 128)
    b3r = b3.reshape(1, 1)
    batch2 = batch.reshape(1, _N)

    h_enc, m = _encode(x, W_encp, b_encp, wpad[0])
    h = h_enc
    for i in range(_L):
        agg2 = _segsum(m, idx, zc)
        h, m = _gru(agg2, h, wir, wiz, win, whr, whz, whn,
                    bir, biz, bin_, bhr, bhz, bhn, wpad[(i + 1) % _L])
    return _pool(h, h_enc, batch2, W1a, W1b, b1r, W2, b2r, W3, b3r)


# speedup vs baseline: 1.0633x; 1.0633x over previous
import functools

import jax
import jax.numpy as jnp
from jax import lax
from jax.experimental import pallas as pl
from jax.experimental.pallas import tpu as pltpu
from jax.experimental.pallas import tpu_sc as plsc

_N = 10000
_E = 320000
_DIN = 128
_C = 101
_OUT = 200
_L = 6
_G = 256
_F = 256
_BLK = 2000
_K = 96
_CHUNKS0 = 186
_CHUNKS1 = 24
_NCK = 16 * (_CHUNKS0 + _CHUNKS1)
_EPAD = _NCK * _K
_AROWS = _N + 16
_RPT = _N // 16


_sc_mesh = plsc.VectorSubcoreMesh(core_axis_name="c", subcore_axis_name="s")


@functools.partial(
    pl.kernel,
    out_type=jax.ShapeDtypeStruct((2, _N, _F), jnp.bfloat16),
    mesh=_sc_mesh,
    compiler_params=pltpu.CompilerParams(use_tc_tiling_on_sc=False),
    scratch_types=[
        [[pltpu.VMEM((2, _K), jnp.int32)] * 2] * 3,
        [pltpu.VMEM((_K, _F), jnp.bfloat16)] * 3,
        pltpu.VMEM_SHARED((_AROWS, _F), jnp.bfloat16),
        [[pltpu.SemaphoreType.DMA] * 2] * 3,
        [pltpu.SemaphoreType.DMA] * 3,
    ],
)
def _segsum(m_hbm, idx_hbm, zc_hbm, out_hbm, ib, rows, agg_sh, si, sg):
    cid = lax.axis_index("c")
    sid = lax.axis_index("s")

    def _prologue(c0):
        for j in range(6):
            b, tp = j % 3, j // 3
            pltpu.async_copy(idx_hbm.at[c0 + j], ib[b][tp], si[b][tp])
        for j in range(3):
            pltpu.make_async_copy(idx_hbm.at[c0 + j], ib[j][0],
                                  si[j][0]).wait()
            pltpu.async_copy(m_hbm.at[ib[j][0].at[0]], rows[j], sg[j])

    def _steady(c, b, tp):
        ntp = 1 - tp
        pltpu.make_async_copy(m_hbm.at[ib[b][tp].at[0]], rows[b],
                              sg[b]).wait()
        pltpu.sync_copy(rows[b], agg_sh.at[ib[b][tp].at[1]], add=True)
        pltpu.async_copy(idx_hbm.at[c + 6], ib[b][tp], si[b][tp])
        pltpu.make_async_copy(idx_hbm.at[c + 3], ib[b][ntp],
                              si[b][ntp]).wait()
        pltpu.async_copy(m_hbm.at[ib[b][ntp].at[0]], rows[b], sg[b])

    def _drain(c0, nchunks):
        def _ring(t, carry):
            base = c0 + t * 6
            for j in range(6):
                _steady(base + j, j % 3, j // 3)
            return carry

        lax.fori_loop(0, (nchunks - 6) // 6, _ring, 0)
        for j in range(3):
            c = c0 + nchunks - 6 + j
            pltpu.make_async_copy(m_hbm.at[ib[j][0].at[0]], rows[j],
                                  sg[j]).wait()
            pltpu.sync_copy(rows[j], agg_sh.at[ib[j][0].at[1]], add=True)
            pltpu.make_async_copy(idx_hbm.at[c + 3], ib[j][1],
                                  si[j][1]).wait()
            pltpu.async_copy(m_hbm.at[ib[j][1].at[0]], rows[j], sg[j])
        for j in range(3):
            pltpu.make_async_copy(m_hbm.at[ib[j][1].at[0]], rows[j],
                                  sg[j]).wait()
            pltpu.sync_copy(rows[j], agg_sh.at[ib[j][1].at[1]], add=True)

    @pl.when(cid == 0)
    def _():
        _prologue(sid * _CHUNKS0)

    @pl.when(cid == 1)
    def _():
        _prologue(16 * _CHUNKS0 + sid * _CHUNKS1)

    r0 = sid * _RPT
    pltpu.sync_copy(zc_hbm, agg_sh.at[pl.ds(r0, _RPT)])
    plsc.subcore_barrier()

    @pl.when(cid == 0)
    def _():
        _drain(sid * _CHUNKS0, _CHUNKS0)

    @pl.when(cid == 1)
    def _():
        _drain(16 * _CHUNKS0 + sid * _CHUNKS1, _CHUNKS1)

    plsc.subcore_barrier()
    pltpu.sync_copy(agg_sh.at[pl.ds(r0, _RPT)],
                    out_hbm.at[cid, pl.ds(r0, _RPT), :])


def _encode_body(x_ref, we_ref, be_ref, w0_ref, h_ref, m0_ref):
    h = jnp.maximum(
        jnp.dot(x_ref[...], we_ref[...], preferred_element_type=jnp.float32)
        + be_ref[...], 0.0)
    h_ref[...] = h
    m = jnp.dot(h, w0_ref[...], preferred_element_type=jnp.float32)
    m0_ref[...] = m.astype(jnp.bfloat16)


_encode = pl.pallas_call(
    _encode_body,
    grid=(_N // _BLK,),
    in_specs=[
        pl.BlockSpec((_BLK, _DIN), lambda i: (i, 0)),
        pl.BlockSpec((_DIN, _OUT), lambda i: (0, 0)),
        pl.BlockSpec((1, _OUT), lambda i: (0, 0)),
        pl.BlockSpec((_OUT, _F), lambda i: (0, 0)),
    ],
    out_specs=[
        pl.BlockSpec((_BLK, _OUT), lambda i: (i, 0)),
        pl.BlockSpec((_BLK, _F), lambda i: (i, 0)),
    ],
    out_shape=[
        jax.ShapeDtypeStruct((_N, _OUT), jnp.float32),
        jax.ShapeDtypeStruct((_N, _F), jnp.bfloat16),
    ],
)


def _gru_body(a_ref, h_ref, wir, wiz, win, whr, whz, whn,
              bir, biz, bin_, bhr, bhz, bhn, wnx, ho_ref, mo_ref):
    f32 = jnp.float32
    av = a_ref[...]
    a = av[0].astype(f32) + av[1].astype(f32)
    h = h_ref[...]
    i_r = jnp.dot(a, wir[...], preferred_element_type=f32) + bir[...]
    i_z = jnp.dot(a, wiz[...], preferred_element_type=f32) + biz[...]
    i_n = jnp.dot(a, win[...], preferred_element_type=f32) + bin_[...]
    h_r = jnp.dot(h, whr[...], preferred_element_type=f32) + bhr[...]
    h_z = jnp.dot(h, whz[...], preferred_element_type=f32) + bhz[...]
    h_n = jnp.dot(h, whn[...], preferred_element_type=f32) + bhn[...]
    r = jax.nn.sigmoid(i_r + h_r)
    z = jax.nn.sigmoid(i_z + h_z)
    n = jnp.tanh(i_n + r * h_n)
    hn = (1.0 - z) * n + z * h
    ho_ref[...] = hn
    mn = jnp.dot(hn, wnx[...], preferred_element_type=f32)
    mo_ref[...] = mn.astype(jnp.bfloat16)


_gru = pl.pallas_call(
    _gru_body,
    grid=(_N // _BLK,),
    in_specs=[
        pl.BlockSpec((2, _BLK, _F), lambda i: (0, i, 0)),
        pl.BlockSpec((_BLK, _OUT), lambda i: (i, 0)),
    ] + [pl.BlockSpec((_F, _OUT), lambda i: (0, 0))] * 3
      + [pl.BlockSpec((_OUT, _OUT), lambda i: (0, 0))] * 3
      + [pl.BlockSpec((1, _OUT), lambda i: (0, 0))] * 6
      + [pl.BlockSpec((_OUT, _F), lambda i: (0, 0))],
    out_specs=[
        pl.BlockSpec((_BLK, _OUT), lambda i: (i, 0)),
        pl.BlockSpec((_BLK, _F), lambda i: (i, 0)),
    ],
    out_shape=[
        jax.ShapeDtypeStruct((_N, _OUT), jnp.float32),
        jax.ShapeDtypeStruct((_N, _F), jnp.bfloat16),
    ],
)


def _pool_body(h_ref, hc_ref, b_ref, w1a, w1b, b1, w2, b2, w3, b3, o_ref):
    f32 = jnp.float32
    bvec = b_ref[...]
    gids = lax.broadcasted_iota(jnp.int32, (_G, _N), 0)
    onehot = jnp.where(bvec == gids, 1.0, 0.0).astype(f32)
    cnt = jnp.sum(onehot, axis=1, keepdims=True)
    inv = 1.0 / jnp.maximum(cnt, 1.0)
    gr_h = jnp.dot(onehot, h_ref[...], preferred_element_type=f32) * inv
    gr_c = jnp.dot(onehot, hc_ref[...], preferred_element_type=f32) * inv
    h1 = jnp.maximum(
        jnp.dot(gr_h, w1a[...], preferred_element_type=f32)
        + jnp.dot(gr_c, w1b[...], preferred_element_type=f32)
        + b1[...], 0.0)
    h2 = jnp.maximum(jnp.dot(h1, w2[...], preferred_element_type=f32)
                     + b2[...], 0.0)
    o_ref[...] = jax.nn.sigmoid(
        jnp.dot(h2, w3[...], preferred_element_type=f32) + b3[...])


_pool = pl.pallas_call(
    _pool_body,
    in_specs=[
        pl.BlockSpec((_N, _OUT), lambda: (0, 0)),
        pl.BlockSpec((_N, _OUT), lambda: (0, 0)),
        pl.BlockSpec((1, _N), lambda: (0, 0)),
        pl.BlockSpec((_OUT, 256), lambda: (0, 0)),
        pl.BlockSpec((_OUT, 256), lambda: (0, 0)),
        pl.BlockSpec((1, 256), lambda: (0, 0)),
        pl.BlockSpec((256, 128), lambda: (0, 0)),
        pl.BlockSpec((1, 128), lambda: (0, 0)),
        pl.BlockSpec((128, 1), lambda: (0, 0)),
        pl.BlockSpec((1, 1), lambda: (0, 0)),
    ],
    out_specs=pl.BlockSpec((_G, 1), lambda: (0, 0)),
    out_shape=jax.ShapeDtypeStruct((_G, 1), jnp.float32),
)


def kernel(x, edge_index, batch, W_enc, b_enc, weight, W_ih, W_hh,
           b_ih, b_hh, W1, b1, W2, b2, W3, b3):
    f32 = jnp.float32
    src = edge_index[0]
    dst = edge_index[1]
    zc = jnp.zeros((_RPT, _F), jnp.bfloat16)
    npad = _EPAD - _E
    srcp = jnp.concatenate([src, jnp.zeros((npad,), jnp.int32)])
    dstp = jnp.concatenate([dst, jnp.full((npad,), _N, jnp.int32)])
    nck = _EPAD // _K
    idx = jnp.stack([srcp.reshape(nck, _K), dstp.reshape(nck, _K)], axis=1)

    W_encp = jnp.pad(W_enc, ((0, 0), (0, _OUT - _C)))
    b_encp = jnp.pad(b_enc, (0, _OUT - _C)).reshape(1, _OUT)
    wpad = jnp.pad(weight, ((0, 0), (0, 0), (0, _F - _OUT)))
    ihT = W_ih.T
    hhT = W_hh.T
    pad_f = ((0, _F - _OUT), (0, 0))
    wir = jnp.pad(ihT[:, 0 * _OUT:1 * _OUT], pad_f)
    wiz = jnp.pad(ihT[:, 1 * _OUT:2 * _OUT], pad_f)
    win = jnp.pad(ihT[:, 2 * _OUT:3 * _OUT], pad_f)
    whr = hhT[:, 0 * _OUT:1 * _OUT]
    whz = hhT[:, 1 * _OUT:2 * _OUT]
    whn = hhT[:, 2 * _OUT:3 * _OUT]
    bir = b_ih[0 * _OUT:1 * _OUT].reshape(1, _OUT)
    biz = b_ih[1 * _OUT:2 * _OUT].reshape(1, _OUT)
    bin_ = b_ih[2 * _OUT:3 * _OUT].reshape(1, _OUT)
    bhr = b_hh[0 * _OUT:1 * _OUT].reshape(1, _OUT)
    bhz = b_hh[1 * _OUT:2 * _OUT].reshape(1, _OUT)
    bhn = b_hh[2 * _OUT:3 * _OUT].reshape(1, _OUT)
    W1a = W1[:_OUT]
    W1b = jnp.pad(W1[_OUT:], ((0, 2 * _OUT - W1.shape[0]), (0, 0)))
    b1r = b1.reshape(1, 256)
    b2r = b2.reshape(1, 128)
    b3r = b3.reshape(1, 1)
    batch2 = batch.reshape(1, _N)

    h_enc, m = _encode(x, W_encp, b_encp, wpad[0])
    h = h_enc
    for i in range(_L):
        agg2 = _segsum(m, idx, zc)
        h, m = _gru(agg2, h, wir, wiz, win, whr, whz, whn,
                    bir, biz, bin_, bhr, bhz, bhn, wpad[(i + 1) % _L])
    return _pool(h, h_enc, batch2, W1a, W1b, b1r, W2, b2r, W3, b3r)
